# baseline (speedup 1.0000x reference)
import functools

import jax
import jax.numpy as jnp
from jax.experimental import pallas as pl
from jax.experimental.pallas import tpu as pltpu


def _se_tile(x_ref, w1_ref, w2_ref, o_ref, *, inv_hw):
    xs = x_ref[...]

    pooled = jnp.sum(xs, axis=-1, dtype=jnp.float32) * inv_hw

    dn = (((1,), (1,)), ((), ()))
    hid = jax.lax.dot_general(pooled, w1_ref[...], dn,
                              preferred_element_type=jnp.float32)
    hid = jnp.maximum(hid, 0.0)
    gate = jax.lax.dot_general(hid, w2_ref[...], dn,
                               preferred_element_type=jnp.float32)
    gate = jax.nn.sigmoid(gate).astype(xs.dtype)

    o_ref[...] = (xs * gate[:, :, None]).astype(o_ref.dtype)


def _pick_batch_tile(B, per_batch_bytes, budget_bytes):
    fits = [t for t in range(1, B + 1)
            if B % t == 0 and t * per_batch_bytes <= budget_bytes]
    if not fits:
        return 1
    even = [t for t in fits if (B // t) % 2 == 0]
    return max(even) if even else max(fits)


def kernel(x, w1, w2):
    B, C, H, W = x.shape
    HW = H * W
    itemsize = jnp.dtype(x.dtype).itemsize
    per_batch = C * HW * itemsize

    x3 = x.reshape(B, C, HW)

    tb = _pick_batch_tile(B, per_batch, 4 << 20)
    grid = (B // tb,)

    slab = tb * per_batch
    weight_bytes = (w1.size + w2.size) * 4
    vmem_limit = int(min(4 * slab + 2 * weight_bytes + (2 << 20), 48 << 20))
    vmem_limit = max(vmem_limit, 16 << 20)

    out = pl.pallas_call(
        functools.partial(_se_tile, inv_hw=float(1.0 / HW)),
        out_shape=jax.ShapeDtypeStruct((B, C, HW), x.dtype),
        grid=grid,
        in_specs=[
            pl.BlockSpec((tb, C, HW), lambda b: (b, 0, 0)),
            pl.BlockSpec(w1.shape, lambda b: (0, 0)),
            pl.BlockSpec(w2.shape, lambda b: (0, 0)),
        ],
        out_specs=pl.BlockSpec((tb, C, HW), lambda b: (b, 0, 0)),
        compiler_params=pltpu.CompilerParams(
            dimension_semantics=("parallel",),
            vmem_limit_bytes=vmem_limit),
    )(x3, w1, w2)
    return out.reshape(B, C, H, W)
